# baseline (speedup 1.0000x reference)
import functools

import jax
import jax.numpy as jnp
import numpy as np
from jax import lax
from jax.experimental import pallas as pl
from jax.experimental.pallas import tpu as pltpu
from jax.experimental.pallas import tpu_sc as plsc

_N_BINS = 18
_DIST_MIN = 3.375
_DIST_MAX = 21.375
_BIN_WID = (_DIST_MAX - _DIST_MIN) / _N_BINS

_THRESHOLDS = [
    float(np.float32((_DIST_MIN + b * _BIN_WID) ** 2)) for b in range(1, _N_BINS)
]

_NC = 2
_NS = 16
_NW = _NC * _NS
_LANES = 16

_CHUNK_J = 256
_TEC_J = 160


def _sc_encode(coords, cmsk, table1d, table2d, *, n, l, d):
    rows_per_w = (n * l) // _NW
    chunks_per_row = l // _CHUNK_J
    n_chunks = rows_per_w * chunks_per_row
    assert n_chunks % 2 == 0
    tec_j = _TEC_J
    str_j = _CHUNK_J - tec_j
    assert tec_j % _LANES == 0 and 0 < str_j <= 128

    mesh = plsc.VectorSubcoreMesh(core_axis_name="c", subcore_axis_name="s")

    @functools.partial(
        pl.kernel,
        out_type=jax.ShapeDtypeStruct((n * l * l, d), jnp.float32),
        mesh=mesh,
        compiler_params=pltpu.CompilerParams(needs_layout_passes=False),
        scratch_types=[
            pltpu.VMEM((3 * n * l,), jnp.float32),
            pltpu.VMEM((n * l,), jnp.float32),
            pltpu.VMEM(((_N_BINS + 1) * d,), jnp.float32),
            pltpu.VMEM((tec_j, d), jnp.float32),
            pltpu.VMEM((tec_j, d), jnp.float32),
            pltpu.VMEM((str_j, d), jnp.float32),
            pltpu.VMEM((str_j, d), jnp.float32),
            pltpu.VMEM((str_j,), jnp.int32),
            pltpu.VMEM((str_j,), jnp.int32),
            pltpu.SemaphoreType.DMA,
            pltpu.SemaphoreType.DMA,
            pltpu.SemaphoreType.DMA,
            pltpu.SemaphoreType.DMA,
            pltpu.SemaphoreType.DMA,
            pltpu.SemaphoreType.DMA,
        ],
    )
    def kern(coords_hbm, cmsk_hbm, table1_hbm, table2_hbm, out_hbm,
             coords_v, cmsk_v, table_v, buft_a, buft_b, bufg_a, bufg_b,
             idx_a, idx_b, semg_a, semg_b, semt_a, semt_b, semo_a, semo_b):
        wid = lax.axis_index("s") * _NC + lax.axis_index("c")
        pltpu.sync_copy(coords_hbm, coords_v)
        pltpu.sync_copy(cmsk_hbm, cmsk_v)
        pltpu.sync_copy(table1_hbm, table_v)

        row0 = wid * rows_per_w
        iota = lax.iota(jnp.int32, _LANES)
        one_i = jnp.full((_LANES,), 1, jnp.int32)
        zero_i = jnp.full((_LANES,), 0, jnp.int32)
        msk_i = jnp.full((_LANES,), _N_BINS, jnp.int32)
        col_offs = [iota + c * _LANES for c in range(d // _LANES)]

        def do_chunk(it, ci, buft, bufg, idx, semg, semt, semo):
            r = row0 + ci // chunks_per_row
            ni = r // l
            ri = r % l
            jbase = (ci % chunks_per_row) * _CHUNK_J
            out_row = r * l + jbase
            xb = ni * 3 * l
            yb = xb + l
            zb = yb + l
            mb = ni * l
            xi = plsc.load_gather(coords_v, [jnp.full((_LANES,), xb + ri, jnp.int32)])
            yi = plsc.load_gather(coords_v, [jnp.full((_LANES,), yb + ri, jnp.int32)])
            zi = plsc.load_gather(coords_v, [jnp.full((_LANES,), zb + ri, jnp.int32)])
            mi = plsc.load_gather(cmsk_v, [jnp.full((_LANES,), mb + ri, jnp.int32)])

            def bins(g):
                js = jbase + g * _LANES
                dx = coords_v[pl.ds(xb + js, _LANES)] - xi
                dy = coords_v[pl.ds(yb + js, _LANES)] - yi
                dz = coords_v[pl.ds(zb + js, _LANES)] - zi
                d2 = dx * dx + dy * dy + dz * dz
                cnt = zero_i
                for thr in _THRESHOLDS:
                    cnt = cnt + jnp.where(d2 >= thr, one_i, zero_i)
                mj = cmsk_v[pl.ds(mb + js, _LANES)] * mi
                return jnp.where(mj > 0.0, cnt, msk_i)

            @plsc.parallel_loop(0, str_j // _LANES)
            def _sgroup(g):
                idx[pl.ds(g * _LANES, _LANES)] = bins(tec_j // _LANES + g)
            gather = pltpu.async_copy(table2_hbm.at[idx], bufg, semg)

            @plsc.parallel_loop(0, tec_j // _LANES)
            def _tgroup(g):
                tbase = bins(g) * d
                for j in range(_LANES):
                    tj = jnp.take_along_axis(
                        tbase, jnp.full((_LANES,), j, jnp.int32), axis=0,
                        mode="promise_in_bounds",
                    )
                    jr = g * _LANES + j
                    for c in range(d // _LANES):
                        v = plsc.load_gather(table_v, [tj + col_offs[c]])
                        buft[jr, pl.ds(c * _LANES, _LANES)] = v

            pltpu.async_copy(buft, out_hbm.at[pl.ds(out_row, tec_j)], semt)
            gather.wait()
            pltpu.async_copy(bufg, out_hbm.at[pl.ds(out_row + tec_j, str_j)], semo)

        def drain_t(buf, sem):
            pltpu.make_async_copy(buf, out_hbm.at[pl.ds(0, tec_j)], sem).wait()

        def drain_o(buf, sem):
            pltpu.make_async_copy(buf, out_hbm.at[pl.ds(0, str_j)], sem).wait()

        @pl.loop(0, n_chunks // 2)
        def _main(it):
            for p, buft, bufg, idx, semg, semt, semo in (
                (0, buft_a, bufg_a, idx_a, semg_a, semt_a, semo_a),
                (1, buft_b, bufg_b, idx_b, semg_b, semt_b, semo_b),
            ):
                @pl.when(it > 0)
                def _():
                    drain_t(buft, semt)
                    drain_o(bufg, semo)
                do_chunk(it, it * 2 + p, buft, bufg, idx, semg, semt, semo)

        drain_t(buft_a, semt_a)
        drain_o(bufg_a, semo_a)
        drain_t(buft_b, semt_b)
        drain_o(bufg_b, semo_b)

    return kern(coords, cmsk, table1d, table2d)


def kernel(cord_tns, cmsk_tns, embed_weight):
    n, l, _, _ = cord_tns.shape
    d = embed_weight.shape[1]
    cord = cord_tns[:, :, 1, :]
    cmsk = cmsk_tns[:, :, 1]
    coords = jnp.transpose(cord, (0, 2, 1)).reshape(3 * n * l)
    cmsk = cmsk.reshape(n * l)
    table2d = jnp.concatenate(
        [embed_weight, jnp.zeros((1, d), jnp.float32)], axis=0
    )
    table1d = table2d.reshape(-1)
    out = _sc_encode(coords, cmsk, table1d, table2d, n=n, l=l, d=d)
    return out.reshape(n, l, l, d)


# speedup vs baseline: 35.8498x; 35.8498x over previous
import functools

import jax
import jax.numpy as jnp
import numpy as np
from jax import lax
from jax.experimental import pallas as pl
from jax.experimental.pallas import tpu as pltpu
from jax.experimental.pallas import tpu_sc as plsc

_N_BINS = 18
_DIST_MIN = 3.375
_DIST_MAX = 21.375
_BIN_WID = (_DIST_MAX - _DIST_MIN) / _N_BINS

_THRESHOLDS = [
    float(np.float32((_DIST_MIN + b * _BIN_WID) ** 2)) for b in range(1, _N_BINS)
]

_NC = 2
_NS = 16
_NW = _NC * _NS
_LANES = 16

_CHUNK_J = 256


def _sc_encode(coords, cmsk, table, *, n, l, d):
    rows_per_w = (n * l) // _NW
    chunks_per_row = l // _CHUNK_J
    n_chunks = rows_per_w * chunks_per_row
    assert n_chunks % 2 == 0

    mesh = plsc.VectorSubcoreMesh(core_axis_name="c", subcore_axis_name="s")

    @functools.partial(
        pl.kernel,
        out_type=jax.ShapeDtypeStruct((n * l * l, d), jnp.float32),
        mesh=mesh,
        compiler_params=pltpu.CompilerParams(needs_layout_passes=False),
        scratch_types=[
            pltpu.VMEM((3 * n * l,), jnp.float32),
            pltpu.VMEM((n * l,), jnp.float32),
            pltpu.VMEM(((_N_BINS + 1) * d,), jnp.float32),
            pltpu.VMEM((_CHUNK_J, d), jnp.float32),
            pltpu.VMEM((_CHUNK_J, d), jnp.float32),
            pltpu.SemaphoreType.DMA,
            pltpu.SemaphoreType.DMA,
        ],
    )
    def kern(coords_hbm, cmsk_hbm, table_hbm, out_hbm,
             coords_v, cmsk_v, table_v, buf_a, buf_b, sem_a, sem_b):
        wid = lax.axis_index("s") * _NC + lax.axis_index("c")
        pltpu.sync_copy(coords_hbm, coords_v)
        pltpu.sync_copy(cmsk_hbm, cmsk_v)
        pltpu.sync_copy(table_hbm, table_v)

        row0 = wid * rows_per_w
        iota = lax.iota(jnp.int32, _LANES)
        one_i = jnp.full((_LANES,), 1, jnp.int32)
        zero_i = jnp.full((_LANES,), 0, jnp.int32)
        msk_i = jnp.full((_LANES,), _N_BINS, jnp.int32)
        col_offs = [iota + c * _LANES for c in range(d // _LANES)]

        def fill_chunk(ci, buf):
            r = row0 + ci // chunks_per_row
            ni = r // l
            ri = r % l
            jbase = (ci % chunks_per_row) * _CHUNK_J
            xb = ni * 3 * l
            yb = xb + l
            zb = yb + l
            mb = ni * l
            xi = plsc.load_gather(coords_v, [jnp.full((_LANES,), xb + ri, jnp.int32)])
            yi = plsc.load_gather(coords_v, [jnp.full((_LANES,), yb + ri, jnp.int32)])
            zi = plsc.load_gather(coords_v, [jnp.full((_LANES,), zb + ri, jnp.int32)])
            mi = plsc.load_gather(cmsk_v, [jnp.full((_LANES,), mb + ri, jnp.int32)])

            @plsc.parallel_loop(0, _CHUNK_J // _LANES)
            def _group(g):
                js = jbase + g * _LANES
                dx = coords_v[pl.ds(xb + js, _LANES)] - xi
                dy = coords_v[pl.ds(yb + js, _LANES)] - yi
                dz = coords_v[pl.ds(zb + js, _LANES)] - zi
                d2 = dx * dx + dy * dy + dz * dz
                cnt = zero_i
                for thr in _THRESHOLDS:
                    cnt = cnt + jnp.where(d2 >= thr, one_i, zero_i)
                mj = cmsk_v[pl.ds(mb + js, _LANES)] * mi
                cnt = jnp.where(mj > 0.0, cnt, msk_i)
                tbase = cnt * d
                for j in range(_LANES):
                    tj = jnp.take_along_axis(
                        tbase, jnp.full((_LANES,), j, jnp.int32), axis=0,
                        mode="promise_in_bounds",
                    )
                    jr = g * _LANES + j
                    for c in range(d // _LANES):
                        v = plsc.load_gather(table_v, [tj + col_offs[c]])
                        buf[jr, pl.ds(c * _LANES, _LANES)] = v
            return r * l + jbase

        def drain(buf, sem):
            pltpu.make_async_copy(buf, out_hbm.at[pl.ds(0, _CHUNK_J)], sem).wait()

        @pl.loop(0, n_chunks // 2)
        def _main(it):
            for p, buf, sem in ((0, buf_a, sem_a), (1, buf_b, sem_b)):
                @pl.when(it > 0)
                def _():
                    drain(buf, sem)
                out_row = fill_chunk(it * 2 + p, buf)
                pltpu.async_copy(buf, out_hbm.at[pl.ds(out_row, _CHUNK_J)], sem)

        drain(buf_a, sem_a)
        drain(buf_b, sem_b)

    return kern(coords, cmsk, table)


def kernel(cord_tns, cmsk_tns, embed_weight):
    n, l, _, _ = cord_tns.shape
    d = embed_weight.shape[1]
    cord = cord_tns[:, :, 1, :]
    cmsk = cmsk_tns[:, :, 1]
    coords = jnp.transpose(cord, (0, 2, 1)).reshape(3 * n * l)
    cmsk = cmsk.reshape(n * l)
    table = jnp.concatenate(
        [embed_weight, jnp.zeros((1, d), jnp.float32)], axis=0
    ).reshape(-1)
    out = _sc_encode(coords, cmsk, table, n=n, l=l, d=d)
    return out.reshape(n, l, l, d)


# speedup vs baseline: 54.2422x; 1.5130x over previous
import functools

import jax
import jax.numpy as jnp
import numpy as np
from jax import lax
from jax.experimental import pallas as pl
from jax.experimental.pallas import tpu as pltpu
from jax.experimental.pallas import tpu_sc as plsc

_N_BINS = 18
_DIST_MIN = 3.375
_DIST_MAX = 21.375
_BIN_WID = (_DIST_MAX - _DIST_MIN) / _N_BINS

_THRESHOLDS = [
    float(np.float32((_DIST_MIN + b * _BIN_WID) ** 2)) for b in range(1, _N_BINS)
]

_NC = 2
_NS = 16
_NW = _NC * _NS
_LANES = 16

_CHUNK_J = 256


def _sc_encode(coords, cmsk, table, *, n, l, d):
    rows_per_w = (n * l) // _NW
    chunks_per_row = l // _CHUNK_J
    n_chunks = rows_per_w * chunks_per_row
    assert n_chunks % 2 == 0

    mesh = plsc.VectorSubcoreMesh(core_axis_name="c", subcore_axis_name="s")

    @functools.partial(
        pl.kernel,
        out_type=jax.ShapeDtypeStruct((n * l * l, d), jnp.float32),
        mesh=mesh,
        compiler_params=pltpu.CompilerParams(needs_layout_passes=False),
        scratch_types=[
            pltpu.VMEM((3 * n * l,), jnp.float32),
            pltpu.VMEM((n * l,), jnp.float32),
            pltpu.VMEM(((_N_BINS + 1) * d,), jnp.float32),
            pltpu.VMEM((_CHUNK_J, d), jnp.float32),
            pltpu.VMEM((_CHUNK_J, d), jnp.float32),
            pltpu.SemaphoreType.DMA,
            pltpu.SemaphoreType.DMA,
        ],
    )
    def kern(coords_hbm, cmsk_hbm, table_hbm, out_hbm,
             coords_v, cmsk_v, table_v, buf_a, buf_b, sem_a, sem_b):
        wid = lax.axis_index("s") * _NC + lax.axis_index("c")
        pltpu.sync_copy(coords_hbm, coords_v)
        pltpu.sync_copy(cmsk_hbm, cmsk_v)
        pltpu.sync_copy(table_hbm, table_v)

        row0 = wid * rows_per_w
        iota = lax.iota(jnp.int32, _LANES)
        one_i = jnp.full((_LANES,), 1, jnp.int32)
        zero_i = jnp.full((_LANES,), 0, jnp.int32)
        msk_i = jnp.full((_LANES,), _N_BINS, jnp.int32)
        col_offs = [iota + c * _LANES for c in range(d // _LANES)]

        def fill_chunk(ci, buf):
            r = row0 + ci // chunks_per_row
            ni = r // l
            ri = r % l
            jbase = (ci % chunks_per_row) * _CHUNK_J
            xb = ni * 3 * l
            yb = xb + l
            zb = yb + l
            mb = ni * l
            xi = plsc.load_gather(coords_v, [jnp.full((_LANES,), xb + ri, jnp.int32)])
            yi = plsc.load_gather(coords_v, [jnp.full((_LANES,), yb + ri, jnp.int32)])
            zi = plsc.load_gather(coords_v, [jnp.full((_LANES,), zb + ri, jnp.int32)])
            mi = plsc.load_gather(cmsk_v, [jnp.full((_LANES,), mb + ri, jnp.int32)])

            @plsc.parallel_loop(0, _CHUNK_J // _LANES)
            def _group(g):
                js = jbase + g * _LANES
                dx = coords_v[pl.ds(xb + js, _LANES)] - xi
                dy = coords_v[pl.ds(yb + js, _LANES)] - yi
                dz = coords_v[pl.ds(zb + js, _LANES)] - zi
                d2 = dx * dx + dy * dy + dz * dz
                cnt = zero_i
                for thr in _THRESHOLDS:
                    cnt = cnt + jnp.where(d2 >= thr, one_i, zero_i)
                mj = cmsk_v[pl.ds(mb + js, _LANES)] * mi
                cnt = jnp.where(mj > 0.0, cnt, msk_i)
                tbase = cnt * d
                for j in range(_LANES):
                    tj = jnp.take_along_axis(
                        tbase, jnp.full((_LANES,), j, jnp.int32), axis=0,
                        mode="promise_in_bounds",
                    )
                    jr = g * _LANES + j
                    vs = [
                        plsc.load_gather(table_v, [tj + col_offs[c]])
                        for c in range(d // _LANES)
                    ]
                    for c, v in enumerate(vs):
                        buf[jr, pl.ds(c * _LANES, _LANES)] = v
            return r * l + jbase

        def drain(buf, sem):
            pltpu.make_async_copy(buf, out_hbm.at[pl.ds(0, _CHUNK_J)], sem).wait()

        @pl.loop(0, n_chunks // 2)
        def _main(it):
            for p, buf, sem in ((0, buf_a, sem_a), (1, buf_b, sem_b)):
                @pl.when(it > 0)
                def _():
                    drain(buf, sem)
                out_row = fill_chunk(it * 2 + p, buf)
                pltpu.async_copy(buf, out_hbm.at[pl.ds(out_row, _CHUNK_J)], sem)

        drain(buf_a, sem_a)
        drain(buf_b, sem_b)

    return kern(coords, cmsk, table)


def kernel(cord_tns, cmsk_tns, embed_weight):
    n, l, _, _ = cord_tns.shape
    d = embed_weight.shape[1]
    cord = cord_tns[:, :, 1, :]
    cmsk = cmsk_tns[:, :, 1]
    coords = jnp.transpose(cord, (0, 2, 1)).reshape(3 * n * l)
    cmsk = cmsk.reshape(n * l)
    table = jnp.concatenate(
        [embed_weight, jnp.zeros((1, d), jnp.float32)], axis=0
    ).reshape(-1)
    out = _sc_encode(coords, cmsk, table, n=n, l=l, d=d)
    return out.reshape(n, l, l, d)


# speedup vs baseline: 59.4827x; 1.0966x over previous
import functools

import jax
import jax.numpy as jnp
import numpy as np
from jax import lax
from jax.experimental import pallas as pl
from jax.experimental.pallas import tpu as pltpu
from jax.experimental.pallas import tpu_sc as plsc

_N_BINS = 18
_DIST_MIN = 3.375
_DIST_MAX = 21.375
_BIN_WID = (_DIST_MAX - _DIST_MIN) / _N_BINS

_THRESHOLDS = [
    float(np.float32((_DIST_MIN + b * _BIN_WID) ** 2)) for b in range(1, _N_BINS)
]

_NC = 2
_NS = 16
_NW = _NC * _NS
_LANES = 16

_CHUNK_J = 256


def _sc_encode(coords, cmsk, table, *, n, l, d):
    rows_per_w = (n * l) // _NW
    chunks_per_row = l // _CHUNK_J
    n_chunks = rows_per_w * chunks_per_row
    assert n_chunks % 2 == 0

    mesh = plsc.VectorSubcoreMesh(core_axis_name="c", subcore_axis_name="s")

    @functools.partial(
        pl.kernel,
        out_type=jax.ShapeDtypeStruct((n * l * l, d), jnp.float32),
        mesh=mesh,
        compiler_params=pltpu.CompilerParams(needs_layout_passes=False),
        scratch_types=[
            pltpu.VMEM((3 * n * l,), jnp.float32),
            pltpu.VMEM((n * l,), jnp.float32),
            pltpu.VMEM(((_N_BINS + 1) * d,), jnp.float32),
            pltpu.VMEM((_CHUNK_J, d), jnp.float32),
            pltpu.VMEM((_CHUNK_J, d), jnp.float32),
            pltpu.SemaphoreType.DMA,
            pltpu.SemaphoreType.DMA,
        ],
    )
    def kern(coords_hbm, cmsk_hbm, table_hbm, out_hbm,
             coords_v, cmsk_v, table_v, buf_a, buf_b, sem_a, sem_b):
        wid = lax.axis_index("s") * _NC + lax.axis_index("c")
        pltpu.sync_copy(coords_hbm, coords_v)
        pltpu.sync_copy(cmsk_hbm, cmsk_v)
        pltpu.sync_copy(table_hbm, table_v)

        row0 = wid * rows_per_w
        iota = lax.iota(jnp.int32, _LANES)
        one_i = jnp.full((_LANES,), 1, jnp.int32)
        zero_i = jnp.full((_LANES,), 0, jnp.int32)
        msk_i = jnp.full((_LANES,), _N_BINS, jnp.int32)
        col_offs = [iota + c * _LANES for c in range(d // _LANES)]

        def fill_chunk(ci, buf):
            r = row0 + ci // chunks_per_row
            ni = r // l
            ri = r % l
            jbase = (ci % chunks_per_row) * _CHUNK_J
            xb = ni * 3 * l
            yb = xb + l
            zb = yb + l
            mb = ni * l
            xi = plsc.load_gather(coords_v, [jnp.full((_LANES,), xb + ri, jnp.int32)])
            yi = plsc.load_gather(coords_v, [jnp.full((_LANES,), yb + ri, jnp.int32)])
            zi = plsc.load_gather(coords_v, [jnp.full((_LANES,), zb + ri, jnp.int32)])
            mi = plsc.load_gather(cmsk_v, [jnp.full((_LANES,), mb + ri, jnp.int32)])

            @plsc.parallel_loop(0, _CHUNK_J // _LANES, unroll=2)
            def _group(g):
                js = jbase + g * _LANES
                dx = coords_v[pl.ds(xb + js, _LANES)] - xi
                dy = coords_v[pl.ds(yb + js, _LANES)] - yi
                dz = coords_v[pl.ds(zb + js, _LANES)] - zi
                d2 = dx * dx + dy * dy + dz * dz
                cnt = zero_i
                for thr in _THRESHOLDS:
                    cnt = cnt + jnp.where(d2 >= thr, one_i, zero_i)
                mj = cmsk_v[pl.ds(mb + js, _LANES)] * mi
                cnt = jnp.where(mj > 0.0, cnt, msk_i)
                tbase = cnt * d
                for j in range(_LANES):
                    tj = jnp.take_along_axis(
                        tbase, jnp.full((_LANES,), j, jnp.int32), axis=0,
                        mode="promise_in_bounds",
                    )
                    jr = g * _LANES + j
                    vs = [
                        plsc.load_gather(table_v, [tj + col_offs[c]])
                        for c in range(d // _LANES)
                    ]
                    for c, v in enumerate(vs):
                        buf[jr, pl.ds(c * _LANES, _LANES)] = v
            return r * l + jbase

        def drain(buf, sem):
            pltpu.make_async_copy(buf, out_hbm.at[pl.ds(0, _CHUNK_J)], sem).wait()

        @pl.loop(0, n_chunks // 2)
        def _main(it):
            for p, buf, sem in ((0, buf_a, sem_a), (1, buf_b, sem_b)):
                @pl.when(it > 0)
                def _():
                    drain(buf, sem)
                out_row = fill_chunk(it * 2 + p, buf)
                pltpu.async_copy(buf, out_hbm.at[pl.ds(out_row, _CHUNK_J)], sem)

        drain(buf_a, sem_a)
        drain(buf_b, sem_b)

    return kern(coords, cmsk, table)


def kernel(cord_tns, cmsk_tns, embed_weight):
    n, l, _, _ = cord_tns.shape
    d = embed_weight.shape[1]
    cord = cord_tns[:, :, 1, :]
    cmsk = cmsk_tns[:, :, 1]
    coords = jnp.transpose(cord, (0, 2, 1)).reshape(3 * n * l)
    cmsk = cmsk.reshape(n * l)
    table = jnp.concatenate(
        [embed_weight, jnp.zeros((1, d), jnp.float32)], axis=0
    ).reshape(-1)
    out = _sc_encode(coords, cmsk, table, n=n, l=l, d=d)
    return out.reshape(n, l, l, d)


# speedup vs baseline: 61.3043x; 1.0306x over previous
import functools

import jax
import jax.numpy as jnp
import numpy as np
from jax import lax
from jax.experimental import pallas as pl
from jax.experimental.pallas import tpu as pltpu
from jax.experimental.pallas import tpu_sc as plsc

_N_BINS = 18
_DIST_MIN = 3.375
_DIST_MAX = 21.375
_BIN_WID = (_DIST_MAX - _DIST_MIN) / _N_BINS

_THRESHOLDS = [
    float(np.float32((_DIST_MIN + b * _BIN_WID) ** 2)) for b in range(1, _N_BINS)
]

_NC = 2
_NS = 16
_NW = _NC * _NS
_LANES = 16

_CHUNK_J = 256


def _sc_encode(coords, cmsk, table, *, n, l, d):
    rows_per_w = (n * l) // _NW
    chunks_per_row = l // _CHUNK_J
    n_chunks = rows_per_w * chunks_per_row
    assert n_chunks % 2 == 0

    mesh = plsc.VectorSubcoreMesh(core_axis_name="c", subcore_axis_name="s")

    @functools.partial(
        pl.kernel,
        out_type=jax.ShapeDtypeStruct((n * l * l, d), jnp.float32),
        mesh=mesh,
        compiler_params=pltpu.CompilerParams(needs_layout_passes=False),
        scratch_types=[
            pltpu.VMEM((3 * n * l,), jnp.float32),
            pltpu.VMEM((n * l,), jnp.float32),
            pltpu.VMEM(((_N_BINS + 1) * d,), jnp.float32),
            pltpu.VMEM((_CHUNK_J, d), jnp.float32),
            pltpu.VMEM((_CHUNK_J, d), jnp.float32),
            pltpu.SemaphoreType.DMA,
            pltpu.SemaphoreType.DMA,
        ],
    )
    def kern(coords_hbm, cmsk_hbm, table_hbm, out_hbm,
             coords_v, cmsk_v, table_v, buf_a, buf_b, sem_a, sem_b):
        wid = lax.axis_index("s") * _NC + lax.axis_index("c")
        pltpu.sync_copy(coords_hbm, coords_v)
        pltpu.sync_copy(cmsk_hbm, cmsk_v)
        pltpu.sync_copy(table_hbm, table_v)

        row0 = wid * rows_per_w
        iota = lax.iota(jnp.int32, _LANES)
        one_i = jnp.full((_LANES,), 1, jnp.int32)
        zero_i = jnp.full((_LANES,), 0, jnp.int32)
        msk_i = jnp.full((_LANES,), _N_BINS, jnp.int32)
        col_offs = [iota + c * _LANES for c in range(d // _LANES)]

        def fill_chunk(ci, buf):
            r = row0 + ci // chunks_per_row
            ni = r // l
            ri = r % l
            jbase = (ci % chunks_per_row) * _CHUNK_J
            xb = ni * 3 * l
            yb = xb + l
            zb = yb + l
            mb = ni * l
            xi = plsc.load_gather(coords_v, [jnp.full((_LANES,), xb + ri, jnp.int32)])
            yi = plsc.load_gather(coords_v, [jnp.full((_LANES,), yb + ri, jnp.int32)])
            zi = plsc.load_gather(coords_v, [jnp.full((_LANES,), zb + ri, jnp.int32)])
            mi = plsc.load_gather(cmsk_v, [jnp.full((_LANES,), mb + ri, jnp.int32)])

            @plsc.parallel_loop(0, _CHUNK_J // _LANES, unroll=4)
            def _group(g):
                js = jbase + g * _LANES
                dx = coords_v[pl.ds(xb + js, _LANES)] - xi
                dy = coords_v[pl.ds(yb + js, _LANES)] - yi
                dz = coords_v[pl.ds(zb + js, _LANES)] - zi
                d2 = dx * dx + dy * dy + dz * dz
                cnt = zero_i
                for thr in _THRESHOLDS:
                    cnt = cnt + jnp.where(d2 >= thr, one_i, zero_i)
                mj = cmsk_v[pl.ds(mb + js, _LANES)] * mi
                cnt = jnp.where(mj > 0.0, cnt, msk_i)
                tbase = cnt * d
                for j in range(_LANES):
                    tj = jnp.take_along_axis(
                        tbase, jnp.full((_LANES,), j, jnp.int32), axis=0,
                        mode="promise_in_bounds",
                    )
                    jr = g * _LANES + j
                    vs = [
                        plsc.load_gather(table_v, [tj + col_offs[c]])
                        for c in range(d // _LANES)
                    ]
                    for c, v in enumerate(vs):
                        buf[jr, pl.ds(c * _LANES, _LANES)] = v
            return r * l + jbase

        def drain(buf, sem):
            pltpu.make_async_copy(buf, out_hbm.at[pl.ds(0, _CHUNK_J)], sem).wait()

        @pl.loop(0, n_chunks // 2)
        def _main(it):
            for p, buf, sem in ((0, buf_a, sem_a), (1, buf_b, sem_b)):
                @pl.when(it > 0)
                def _():
                    drain(buf, sem)
                out_row = fill_chunk(it * 2 + p, buf)
                pltpu.async_copy(buf, out_hbm.at[pl.ds(out_row, _CHUNK_J)], sem)

        drain(buf_a, sem_a)
        drain(buf_b, sem_b)

    return kern(coords, cmsk, table)


def kernel(cord_tns, cmsk_tns, embed_weight):
    n, l, _, _ = cord_tns.shape
    d = embed_weight.shape[1]
    cord = cord_tns[:, :, 1, :]
    cmsk = cmsk_tns[:, :, 1]
    coords = jnp.transpose(cord, (0, 2, 1)).reshape(3 * n * l)
    cmsk = cmsk.reshape(n * l)
    table = jnp.concatenate(
        [embed_weight, jnp.zeros((1, d), jnp.float32)], axis=0
    ).reshape(-1)
    out = _sc_encode(coords, cmsk, table, n=n, l=l, d=d)
    return out.reshape(n, l, l, d)
